# Initial kernel scaffold; baseline (speedup 1.0000x reference)
import jax
import jax.numpy as jnp
from jax.experimental import pallas as pl


def kernel(origin_user_embedding, origin_item_embedding, item_side_feat, Wu, bu, Wi, bi, edge_index):
    raise NotImplementedError("write your pallas kernel here")


# speedup vs baseline: 5.0640x; 5.0640x over previous
import functools

import jax
import jax.numpy as jnp
from jax import lax
from jax.experimental import pallas as pl
from jax.experimental.pallas import tpu as pltpu
from jax.experimental.pallas import tpu_sc as plsc

NU = 5000
N = 10000
D = 128
E = 320000
NC = 2
NS = 16
RPB = 128
EP = 327680
RP = EP // RPB
R_T32 = RP // (NC * NS)
NPAD = 10240
RT = NPAD // NS

_sc_mesh = plsc.VectorSubcoreMesh(core_axis_name="c", subcore_axis_name="s")


@functools.partial(
    pl.kernel,
    out_type=jax.ShapeDtypeStruct((NC * NPAD, D), jnp.float32),
    mesh=_sc_mesh,
    scratch_types=(
        pltpu.VMEM((RPB,), jnp.int32),
        pltpu.VMEM((RPB,), jnp.int32),
        pltpu.VMEM((RPB, D), jnp.float32),
        pltpu.VMEM((RPB, D), jnp.float32),
        pltpu.VMEM_SHARED((NPAD, D), jnp.float32),
        pltpu.SemaphoreType.DMA,
    ),
)
def _sc_gcn(t_hbm, src_hbm, dst_hbm, zrow_hbm, out,
            sidx, didx, rows, zer_v, acc, sem):
    c = lax.axis_index("c")
    s = lax.axis_index("s")
    wid = s * NC + c
    pltpu.sync_copy(zrow_hbm, zer_v)
    for z in range(RT // RPB):
        pltpu.sync_copy(zer_v, acc.at[pl.ds(s * RT + z * RPB, RPB)])
    plsc.subcore_barrier()

    def body(g, carry):
        base = (wid * R_T32 + g) * RPB
        pltpu.sync_copy(src_hbm.at[pl.ds(base, RPB)], sidx)
        pltpu.sync_copy(dst_hbm.at[pl.ds(base, RPB)], didx)
        pltpu.async_copy(t_hbm.at[sidx], rows, sem).wait()
        pltpu.sync_copy(rows, acc.at[didx], add=True)
        return carry

    lax.fori_loop(0, R_T32, body, 0)
    plsc.subcore_barrier()
    for z in range(RT // RPB):
        sl = pl.ds(s * RT + z * RPB, RPB)
        pltpu.sync_copy(acc.at[sl], rows)
        pltpu.sync_copy(rows, out.at[pl.ds(c * NPAD + s * RT + z * RPB, RPB)])


def _tc_dense_body(x_ref, wt_ref, b_ref, o_ref):
    y = jnp.dot(x_ref[...], wt_ref[0], preferred_element_type=jnp.float32)
    o_ref[...] = jnp.maximum(y + b_ref[0, 0][None, :], 0.0)


_ROWS_BLK = 1000


def _tc_dense(x, wt, b):
    return pl.pallas_call(
        _tc_dense_body,
        grid=(N // _ROWS_BLK,),
        in_specs=[
            pl.BlockSpec((_ROWS_BLK, D), lambda i: (i, 0)),
            pl.BlockSpec((1, D, D), lambda i: (i // 5, 0, 0)),
            pl.BlockSpec((1, 1, D), lambda i: (i // 5, 0, 0)),
        ],
        out_specs=pl.BlockSpec((_ROWS_BLK, D), lambda i: (i, 0)),
        out_shape=jax.ShapeDtypeStruct((N, D), jnp.float32),
    )(x, wt, b)


def _pad_nodes(t):
    return jnp.pad(t, ((0, NPAD - N), (0, 0)))


def kernel(origin_user_embedding, origin_item_embedding, item_side_feat,
           Wu, bu, Wi, bi, edge_index):
    src = edge_index[0]
    dst = edge_index[1]
    pad = EP - E
    pad_src = (jnp.arange(pad, dtype=jnp.int32) % 128) + N
    pad_dst = (jnp.arange(pad, dtype=jnp.int32) % 112) + N + 128
    srcp = jnp.concatenate([src, pad_src])
    dstp = jnp.concatenate([dst, pad_dst])
    zrow = jnp.zeros((RPB, D), jnp.float32)

    out_deg = jnp.maximum(
        jnp.bincount(src, length=N).astype(jnp.float32), 1.0
    )
    in_deg = jnp.maximum(
        jnp.bincount(dst, length=N).astype(jnp.float32), 1.0
    )
    oi = lax.rsqrt(out_deg)
    ii = lax.rsqrt(in_deg)

    x = jnp.concatenate([origin_user_embedding, origin_item_embedding], axis=0)
    wt = jnp.stack([Wu.T, Wi.T])
    bb = jnp.stack([bu, bi]).reshape(2, 1, D)
    node0 = _tc_dense(x, wt, bb)

    t0 = _pad_nodes(node0 * oi[:, None])
    p0 = _sc_gcn(t0, srcp, dstp, zrow)
    emb0 = (p0[:NPAD] + p0[NPAD:])[:N] * ii[:, None]

    t1 = _pad_nodes(emb0 * oi[:, None])
    p1 = _sc_gcn(t1, srcp, dstp, zrow)
    emb1 = (p1[:NPAD] + p1[NPAD:])[:N] * ii[:, None]

    user = node0[:NU] + emb0[:NU] * 0.5 + emb1[:NU] * (1.0 / 3.0)
    item = (node0[NU:] + emb0[NU:] * 0.5 + emb1[NU:] * (1.0 / 3.0)
            + item_side_feat)
    return (user, item)


# speedup vs baseline: 7.4052x; 1.4623x over previous
import functools

import jax
import jax.numpy as jnp
from jax import lax
from jax.experimental import pallas as pl
from jax.experimental.pallas import tpu as pltpu
from jax.experimental.pallas import tpu_sc as plsc

NU = 5000
N = 10000
D = 128
E = 320000
NC = 2
NS = 16
RPB = 128
EP = 327680
RP = EP // RPB
R_T32 = RP // (NC * NS)
NPAD = 10240
RT = NPAD // NS

_sc_mesh = plsc.VectorSubcoreMesh(core_axis_name="c", subcore_axis_name="s")


@functools.partial(
    pl.kernel,
    out_type=jax.ShapeDtypeStruct((NC * NPAD, D), jnp.float32),
    mesh=_sc_mesh,
    scratch_types=(
        pltpu.VMEM((RPB,), jnp.int32),
        pltpu.VMEM((RPB,), jnp.int32),
        pltpu.VMEM((RPB,), jnp.int32),
        pltpu.VMEM((RPB,), jnp.int32),
        pltpu.VMEM((RPB, D), jnp.float32),
        pltpu.VMEM((RPB, D), jnp.float32),
        pltpu.VMEM_SHARED((NPAD, D), jnp.float32),
        pltpu.SemaphoreType.DMA,
        pltpu.SemaphoreType.DMA,
    ),
)
def _sc_gcn(t_hbm, src_hbm, dst_hbm, zblk_hbm, out,
            sidx0, sidx1, didx0, didx1, rows0, rows1, acc, sem0, sem1):
    c = lax.axis_index("c")
    s = lax.axis_index("s")
    wid = s * NC + c
    base = wid * R_T32 * RPB
    pltpu.sync_copy(zblk_hbm, acc.at[pl.ds(s * RT, RT)])
    plsc.subcore_barrier()

    pltpu.sync_copy(src_hbm.at[pl.ds(base, RPB)], sidx0)
    pltpu.sync_copy(dst_hbm.at[pl.ds(base, RPB)], didx0)
    pltpu.async_copy(t_hbm.at[sidx0], rows0, sem0)
    pltpu.sync_copy(src_hbm.at[pl.ds(base + RPB, RPB)], sidx1)
    pltpu.sync_copy(dst_hbm.at[pl.ds(base + RPB, RPB)], didx1)
    pltpu.async_copy(t_hbm.at[sidx1], rows1, sem1)

    def body(gg, carry):
        b0 = base + gg * (2 * RPB)
        pltpu.make_async_copy(t_hbm.at[sidx0], rows0, sem0).wait()
        pltpu.sync_copy(rows0, acc.at[didx0], add=True)
        pltpu.sync_copy(src_hbm.at[pl.ds(b0 + 2 * RPB, RPB)], sidx0)
        pltpu.sync_copy(dst_hbm.at[pl.ds(b0 + 2 * RPB, RPB)], didx0)
        pltpu.async_copy(t_hbm.at[sidx0], rows0, sem0)
        pltpu.make_async_copy(t_hbm.at[sidx1], rows1, sem1).wait()
        pltpu.sync_copy(rows1, acc.at[didx1], add=True)
        pltpu.sync_copy(src_hbm.at[pl.ds(b0 + 3 * RPB, RPB)], sidx1)
        pltpu.sync_copy(dst_hbm.at[pl.ds(b0 + 3 * RPB, RPB)], didx1)
        pltpu.async_copy(t_hbm.at[sidx1], rows1, sem1)
        return carry

    lax.fori_loop(0, (R_T32 - 2) // 2, body, 0)
    pltpu.make_async_copy(t_hbm.at[sidx0], rows0, sem0).wait()
    pltpu.sync_copy(rows0, acc.at[didx0], add=True)
    pltpu.make_async_copy(t_hbm.at[sidx1], rows1, sem1).wait()
    pltpu.sync_copy(rows1, acc.at[didx1], add=True)

    plsc.subcore_barrier()
    pltpu.sync_copy(acc.at[pl.ds(s * RT, RT)],
                    out.at[pl.ds(c * NPAD + s * RT, RT)])


def _tc_dense_body(x_ref, wt_ref, b_ref, o_ref):
    y = jnp.dot(x_ref[...], wt_ref[0], preferred_element_type=jnp.float32)
    o_ref[...] = jnp.maximum(y + b_ref[0, 0][None, :], 0.0)


_ROWS_BLK = 1000


def _tc_dense(x, wt, b):
    return pl.pallas_call(
        _tc_dense_body,
        grid=(N // _ROWS_BLK,),
        in_specs=[
            pl.BlockSpec((_ROWS_BLK, D), lambda i: (i, 0)),
            pl.BlockSpec((1, D, D), lambda i: (i // 5, 0, 0)),
            pl.BlockSpec((1, 1, D), lambda i: (i // 5, 0, 0)),
        ],
        out_specs=pl.BlockSpec((_ROWS_BLK, D), lambda i: (i, 0)),
        out_shape=jax.ShapeDtypeStruct((N, D), jnp.float32),
    )(x, wt, b)


def _pad_nodes(t):
    return jnp.pad(t, ((0, NPAD - N), (0, 0)))


def kernel(origin_user_embedding, origin_item_embedding, item_side_feat,
           Wu, bu, Wi, bi, edge_index):
    src = edge_index[0]
    dst = edge_index[1]
    pad = EP - E
    pad_src = (jnp.arange(pad, dtype=jnp.int32) % 128) + N
    pad_dst = (jnp.arange(pad, dtype=jnp.int32) % 112) + N + 128
    srcp = jnp.concatenate([src, pad_src])
    dstp = jnp.concatenate([dst, pad_dst])
    zblk = jnp.zeros((RT, D), jnp.float32)

    out_deg = jnp.maximum(
        jnp.bincount(src, length=N).astype(jnp.float32), 1.0
    )
    in_deg = jnp.maximum(
        jnp.bincount(dst, length=N).astype(jnp.float32), 1.0
    )
    oi = lax.rsqrt(out_deg)
    ii = lax.rsqrt(in_deg)

    x = jnp.concatenate([origin_user_embedding, origin_item_embedding], axis=0)
    wt = jnp.stack([Wu.T, Wi.T])
    bb = jnp.stack([bu, bi]).reshape(2, 1, D)
    node0 = _tc_dense(x, wt, bb)

    t0 = _pad_nodes(node0 * oi[:, None])
    p0 = _sc_gcn(t0, srcp, dstp, zblk)
    emb0 = (p0[:NPAD] + p0[NPAD:])[:N] * ii[:, None]

    t1 = _pad_nodes(emb0 * oi[:, None])
    p1 = _sc_gcn(t1, srcp, dstp, zblk)
    emb1 = (p1[:NPAD] + p1[NPAD:])[:N] * ii[:, None]

    user = node0[:NU] + emb0[:NU] * 0.5 + emb1[:NU] * (1.0 / 3.0)
    item = (node0[NU:] + emb0[NU:] * 0.5 + emb1[NU:] * (1.0 / 3.0)
            + item_side_feat)
    return (user, item)


# speedup vs baseline: 9.2872x; 1.2541x over previous
import functools

import jax
import jax.numpy as jnp
from jax import lax
from jax.experimental import pallas as pl
from jax.experimental.pallas import tpu as pltpu
from jax.experimental.pallas import tpu_sc as plsc

NU = 5000
N = 10000
D = 128
E = 320000
NC = 2
NS = 16
RPB = 128
EP = 327680
RP = EP // RPB
R_T32 = RP // (NC * NS)
NPAD = 10240
RT = NPAD // NS

_sc_mesh = plsc.VectorSubcoreMesh(core_axis_name="c", subcore_axis_name="s")


@functools.partial(
    pl.kernel,
    out_type=jax.ShapeDtypeStruct((NC * NPAD, D), jnp.float32),
    mesh=_sc_mesh,
    scratch_types=(
        pltpu.VMEM((RPB,), jnp.int32),
        pltpu.VMEM((RPB,), jnp.int32),
        pltpu.VMEM((RPB,), jnp.int32),
        pltpu.VMEM((RPB,), jnp.int32),
        pltpu.VMEM((RPB,), jnp.int32),
        pltpu.VMEM((RPB,), jnp.int32),
        pltpu.VMEM((RPB,), jnp.int32),
        pltpu.VMEM((RPB,), jnp.int32),
        pltpu.VMEM((RPB, D), jnp.float32),
        pltpu.VMEM((RPB, D), jnp.float32),
        pltpu.VMEM_SHARED((NPAD, D), jnp.float32),
        pltpu.SemaphoreType.DMA,
        pltpu.SemaphoreType.DMA,
        pltpu.SemaphoreType.DMA,
        pltpu.SemaphoreType.DMA,
        pltpu.SemaphoreType.DMA,
        pltpu.SemaphoreType.DMA,
        pltpu.SemaphoreType.DMA,
        pltpu.SemaphoreType.DMA,
        pltpu.SemaphoreType.DMA,
        pltpu.SemaphoreType.DMA,
    ),
)
def _sc_gcn(t_hbm, src_hbm, dst_hbm, zblk_hbm, out,
            sidx0, sidx1, sidx2, sidx3, didx0, didx1, didx2, didx3,
            rows0, rows1, acc,
            ss0, ss1, ss2, ss3, ds0, ds1, ds2, ds3, g0, g1):
    c = lax.axis_index("c")
    s = lax.axis_index("s")
    wid = s * NC + c
    base = wid * R_T32 * RPB
    sidx = (sidx0, sidx1, sidx2, sidx3)
    didx = (didx0, didx1, didx2, didx3)
    rows = (rows0, rows1)
    ssem = (ss0, ss1, ss2, ss3)
    dsem = (ds0, ds1, ds2, ds3)
    gsem = (g0, g1)
    pltpu.sync_copy(zblk_hbm, acc.at[pl.ds(s * RT, RT)])
    plsc.subcore_barrier()

    def issue_idx(koff, q):
        pltpu.async_copy(src_hbm.at[pl.ds(base + koff, RPB)], sidx[q], ssem[q])
        pltpu.async_copy(dst_hbm.at[pl.ds(base + koff, RPB)], didx[q], dsem[q])

    def issue_gather(q, p):
        pltpu.async_copy(t_hbm.at[sidx[q]], rows[p], gsem[p])

    for i in range(4):
        issue_idx(i * RPB, i)
    for i in range(2):
        pltpu.make_async_copy(src_hbm.at[pl.ds(base + i * RPB, RPB)],
                              sidx[i], ssem[i]).wait()
        issue_gather(i, i)

    def step(koff, k_static):
        p = k_static % 2
        q = k_static % 4
        q2 = (k_static + 2) % 4
        pltpu.make_async_copy(t_hbm.at[sidx[q]], rows[p], gsem[p]).wait()
        pltpu.make_async_copy(dst_hbm.at[pl.ds(base + koff, RPB)],
                              didx[q], dsem[q]).wait()
        pltpu.sync_copy(rows[p], acc.at[didx[q]], add=True)
        issue_idx(koff + 4 * RPB, q)
        pltpu.make_async_copy(src_hbm.at[pl.ds(base + koff + 2 * RPB, RPB)],
                              sidx[q2], ssem[q2]).wait()
        issue_gather(q2, p)

    def body(j, carry):
        b0 = j * (4 * RPB)
        for i in range(4):
            step(b0 + i * RPB, i)
        return carry

    NBLK = (R_T32 - 4) // 4
    lax.fori_loop(0, NBLK, body, 0)

    eb = (R_T32 - 4) * RPB
    for i in range(2):
        k = R_T32 - 4 + i
        p, q, q2 = k % 2, k % 4, (k + 2) % 4
        pltpu.make_async_copy(t_hbm.at[sidx[q]], rows[p], gsem[p]).wait()
        pltpu.make_async_copy(dst_hbm.at[pl.ds(eb + i * RPB, RPB)],
                              didx[q], dsem[q]).wait()
        pltpu.sync_copy(rows[p], acc.at[didx[q]], add=True)
        pltpu.make_async_copy(src_hbm.at[pl.ds(eb + (i + 2) * RPB, RPB)],
                              sidx[q2], ssem[q2]).wait()
        issue_gather(q2, p)
    for i in range(2, 4):
        k = R_T32 - 4 + i
        p, q = k % 2, k % 4
        pltpu.make_async_copy(t_hbm.at[sidx[q]], rows[p], gsem[p]).wait()
        pltpu.make_async_copy(dst_hbm.at[pl.ds(eb + i * RPB, RPB)],
                              didx[q], dsem[q]).wait()
        pltpu.sync_copy(rows[p], acc.at[didx[q]], add=True)

    plsc.subcore_barrier()
    pltpu.sync_copy(acc.at[pl.ds(s * RT, RT)],
                    out.at[pl.ds(c * NPAD + s * RT, RT)])


def _tc_dense_body(x_ref, wt_ref, b_ref, o_ref):
    y = jnp.dot(x_ref[...], wt_ref[0], preferred_element_type=jnp.float32)
    o_ref[...] = jnp.maximum(y + b_ref[0, 0][None, :], 0.0)


_ROWS_BLK = 1000


def _tc_dense(x, wt, b):
    return pl.pallas_call(
        _tc_dense_body,
        grid=(N // _ROWS_BLK,),
        in_specs=[
            pl.BlockSpec((_ROWS_BLK, D), lambda i: (i, 0)),
            pl.BlockSpec((1, D, D), lambda i: (i // 5, 0, 0)),
            pl.BlockSpec((1, 1, D), lambda i: (i // 5, 0, 0)),
        ],
        out_specs=pl.BlockSpec((_ROWS_BLK, D), lambda i: (i, 0)),
        out_shape=jax.ShapeDtypeStruct((N, D), jnp.float32),
    )(x, wt, b)


def _pad_nodes(t):
    return jnp.pad(t, ((0, NPAD - N), (0, 0)))


def kernel(origin_user_embedding, origin_item_embedding, item_side_feat,
           Wu, bu, Wi, bi, edge_index):
    src = edge_index[0]
    dst = edge_index[1]
    pad = EP - E
    pad_src = (jnp.arange(pad, dtype=jnp.int32) % 128) + N
    pad_dst = (jnp.arange(pad, dtype=jnp.int32) % 112) + N + 128
    srcp = jnp.concatenate([src, pad_src])
    dstp = jnp.concatenate([dst, pad_dst])
    zblk = jnp.zeros((RT, D), jnp.float32)

    out_deg = jnp.maximum(
        jnp.bincount(src, length=N).astype(jnp.float32), 1.0
    )
    in_deg = jnp.maximum(
        jnp.bincount(dst, length=N).astype(jnp.float32), 1.0
    )
    oi = lax.rsqrt(out_deg)
    ii = lax.rsqrt(in_deg)

    x = jnp.concatenate([origin_user_embedding, origin_item_embedding], axis=0)
    wt = jnp.stack([Wu.T, Wi.T])
    bb = jnp.stack([bu, bi]).reshape(2, 1, D)
    node0 = _tc_dense(x, wt, bb)

    t0 = _pad_nodes(node0 * oi[:, None])
    p0 = _sc_gcn(t0, srcp, dstp, zblk)
    emb0 = (p0[:NPAD] + p0[NPAD:])[:N] * ii[:, None]

    t1 = _pad_nodes(emb0 * oi[:, None])
    p1 = _sc_gcn(t1, srcp, dstp, zblk)
    emb1 = (p1[:NPAD] + p1[NPAD:])[:N] * ii[:, None]

    user = node0[:NU] + emb0[:NU] * 0.5 + emb1[:NU] * (1.0 / 3.0)
    item = (node0[NU:] + emb0[NU:] * 0.5 + emb1[NU:] * (1.0 / 3.0)
            + item_side_feat)
    return (user, item)
